# initial kernel scaffold (unmeasured)
import functools

import jax
import jax.numpy as jnp
from jax import lax
from jax.experimental import pallas as pl
from jax.experimental.pallas import tpu as pltpu

N_DEV = 32


def _ring_allreduce(x, collective_id):
    rows, cols = x.shape
    chunk = rows // N_DEV

    def body(x_ref, o_ref, rs_buf, ag_buf, rs_send, rs_recv, ag_send, ag_recv):
        me = lax.axis_index("i")
        left = lax.rem(me + N_DEV - 1, N_DEV)
        right = lax.rem(me + 1, N_DEV)

        barrier = pltpu.get_barrier_semaphore()
        for nbr in (left, right):
            pl.semaphore_signal(
                barrier, inc=1,
                device_id=(nbr,), device_id_type=pl.DeviceIdType.MESH,
            )
        pl.semaphore_wait(barrier, 2)

        o_ref[...] = x_ref[...]

        for h in range(N_DEV - 1):
            c_send = lax.rem(me - h + 2 * N_DEV, N_DEV)
            rdma = pltpu.make_async_remote_copy(
                src_ref=o_ref.at[pl.ds(c_send * chunk, chunk), :],
                dst_ref=rs_buf.at[h],
                send_sem=rs_send.at[h],
                recv_sem=rs_recv.at[h],
                device_id=(right,),
                device_id_type=pl.DeviceIdType.MESH,
            )
            rdma.start()
            rdma.wait()
            c_recv = lax.rem(me - h - 1 + 2 * N_DEV, N_DEV)
            sl = pl.ds(c_recv * chunk, chunk)
            o_ref[sl, :] = o_ref[sl, :] + rs_buf[h]

        for h in range(N_DEV - 1):
            c_send = lax.rem(me + 1 - h + 2 * N_DEV, N_DEV)
            rdma = pltpu.make_async_remote_copy(
                src_ref=o_ref.at[pl.ds(c_send * chunk, chunk), :],
                dst_ref=ag_buf.at[h],
                send_sem=ag_send.at[h],
                recv_sem=ag_recv.at[h],
                device_id=(right,),
                device_id_type=pl.DeviceIdType.MESH,
            )
            rdma.start()
            rdma.wait()
            c_recv = lax.rem(me - h + 2 * N_DEV, N_DEV)
            o_ref[pl.ds(c_recv * chunk, chunk), :] = ag_buf[h]

    return pl.pallas_call(
        body,
        out_shape=jax.ShapeDtypeStruct((rows, cols), jnp.float32),
        in_specs=[pl.BlockSpec(memory_space=pltpu.VMEM)],
        out_specs=pl.BlockSpec(memory_space=pltpu.VMEM),
        scratch_shapes=[
            pltpu.VMEM((N_DEV - 1, chunk, cols), jnp.float32),
            pltpu.VMEM((N_DEV - 1, chunk, cols), jnp.float32),
            pltpu.SemaphoreType.DMA((N_DEV - 1,)),
            pltpu.SemaphoreType.DMA((N_DEV - 1,)),
            pltpu.SemaphoreType.DMA((N_DEV - 1,)),
            pltpu.SemaphoreType.DMA((N_DEV - 1,)),
        ],
        compiler_params=pltpu.CompilerParams(
            collective_id=collective_id, has_side_effects=True
        ),
    )(x)


def _ln(h):
    m = h.mean(axis=-1, keepdims=True)
    v = h.var(axis=-1, keepdims=True)
    return (h - m) / jnp.sqrt(v + 1e-5)


def kernel(x, Wq, Wk, Wv, Wo, t_emb, W_mod, W_ff1, W_ff2):
    B, S, D = x.shape
    Dh = 96
    Hloc = Wq.shape[1] // Dh

    x0 = x.astype(jnp.float32)
    mod = t_emb @ W_mod
    sa, sha, ga, sm, shm, gm = jnp.split(mod, 6, axis=-1)

    xm = _ln(x0) * (1.0 + sa[:, None, :]) + sha[:, None, :]

    Q = (xm @ Wq).reshape(B, S, Hloc, Dh)
    K = (xm @ Wk).reshape(B, S, Hloc, Dh)
    V = (xm @ Wv).reshape(B, S, Hloc, Dh)
    s = jnp.einsum("bihd,bjhd->bhij", Q, K) * 0.10206207261596577
    p = jax.nn.softmax(s.astype(jnp.float32), axis=-1)
    o = jnp.einsum("bhij,bjhd->bihd", p, V).reshape(B, S, Hloc * Dh)
    attn_partial = o @ Wo

    attn = _ring_allreduce(attn_partial.reshape(B * S, D), 0).reshape(B, S, D)
    x1 = x0 + ga[:, None, :] * attn

    xm2 = _ln(x1) * (1.0 + sm[:, None, :]) + shm[:, None, :]
    h = xm2 @ W_ff1
    h = h * jax.nn.sigmoid(h)
    ff_partial = h @ W_ff2

    ff = _ring_allreduce(ff_partial.reshape(B * S, D), 1).reshape(B, S, D)
    return x1 + gm[:, None, :] * ff


# baseline (device time: 404371 ns/iter reference)
import functools

import jax
import jax.numpy as jnp
from jax import lax
from jax.experimental import pallas as pl
from jax.experimental.pallas import tpu as pltpu

N_DEV = 32


def _ring_allreduce(x, collective_id):
    rows, cols = x.shape
    chunk = rows // N_DEV

    def body(x_ref, o_ref, rs_buf, ag_buf, rs_send, rs_recv, ag_send, ag_recv):
        me = lax.axis_index("i")
        right = lax.rem(me + 1, N_DEV)

        o_ref[...] = x_ref[...]

        for h in range(N_DEV - 1):
            c_send = lax.rem(me - h + 2 * N_DEV, N_DEV)
            rdma = pltpu.make_async_remote_copy(
                src_ref=o_ref.at[pl.ds(c_send * chunk, chunk), :],
                dst_ref=rs_buf.at[h],
                send_sem=rs_send.at[h],
                recv_sem=rs_recv.at[h],
                device_id=(right,),
                device_id_type=pl.DeviceIdType.MESH,
            )
            rdma.start()
            rdma.wait()
            c_recv = lax.rem(me - h - 1 + 2 * N_DEV, N_DEV)
            sl = pl.ds(c_recv * chunk, chunk)
            o_ref[sl, :] = o_ref[sl, :] + rs_buf[h]

        for h in range(N_DEV - 1):
            c_send = lax.rem(me + 1 - h + 2 * N_DEV, N_DEV)
            rdma = pltpu.make_async_remote_copy(
                src_ref=o_ref.at[pl.ds(c_send * chunk, chunk), :],
                dst_ref=ag_buf.at[h],
                send_sem=ag_send.at[h],
                recv_sem=ag_recv.at[h],
                device_id=(right,),
                device_id_type=pl.DeviceIdType.MESH,
            )
            rdma.start()
            rdma.wait()
            c_recv = lax.rem(me - h + 2 * N_DEV, N_DEV)
            o_ref[pl.ds(c_recv * chunk, chunk), :] = ag_buf[h]

    return pl.pallas_call(
        body,
        out_shape=jax.ShapeDtypeStruct((rows, cols), jnp.float32),
        in_specs=[pl.BlockSpec(memory_space=pltpu.VMEM)],
        out_specs=pl.BlockSpec(memory_space=pltpu.VMEM),
        scratch_shapes=[
            pltpu.VMEM((N_DEV - 1, chunk, cols), jnp.float32),
            pltpu.VMEM((N_DEV - 1, chunk, cols), jnp.float32),
            pltpu.SemaphoreType.DMA((N_DEV - 1,)),
            pltpu.SemaphoreType.DMA((N_DEV - 1,)),
            pltpu.SemaphoreType.DMA((N_DEV - 1,)),
            pltpu.SemaphoreType.DMA((N_DEV - 1,)),
        ],
        compiler_params=pltpu.CompilerParams(has_side_effects=True),
    )(x)


def _ln(h):
    m = h.mean(axis=-1, keepdims=True)
    v = h.var(axis=-1, keepdims=True)
    return (h - m) / jnp.sqrt(v + 1e-5)


def kernel(x, Wq, Wk, Wv, Wo, t_emb, W_mod, W_ff1, W_ff2):
    B, S, D = x.shape
    Dh = 96
    Hloc = Wq.shape[1] // Dh

    x0 = x.astype(jnp.float32)
    mod = t_emb @ W_mod
    sa, sha, ga, sm, shm, gm = jnp.split(mod, 6, axis=-1)

    xm = _ln(x0) * (1.0 + sa[:, None, :]) + sha[:, None, :]

    Q = (xm @ Wq).reshape(B, S, Hloc, Dh)
    K = (xm @ Wk).reshape(B, S, Hloc, Dh)
    V = (xm @ Wv).reshape(B, S, Hloc, Dh)
    s = jnp.einsum("bihd,bjhd->bhij", Q, K) * 0.10206207261596577
    p = jax.nn.softmax(s.astype(jnp.float32), axis=-1)
    o = jnp.einsum("bhij,bjhd->bihd", p, V).reshape(B, S, Hloc * Dh)
    attn_partial = o @ Wo

    attn = _ring_allreduce(attn_partial.reshape(B * S, D), 0).reshape(B, S, D)
    x1 = x0 + ga[:, None, :] * attn

    xm2 = _ln(x1) * (1.0 + sm[:, None, :]) + shm[:, None, :]
    h = xm2 @ W_ff1
    h = h * jax.nn.sigmoid(h)
    ff_partial = h @ W_ff2

    ff = _ring_allreduce(ff_partial.reshape(B * S, D), 1).reshape(B, S, D)
    return x1 + gm[:, None, :] * ff


# device time: 289334 ns/iter; 1.3976x vs baseline; 1.3976x over previous
import functools

import jax
import jax.numpy as jnp
from jax import lax
from jax.experimental import pallas as pl
from jax.experimental.pallas import tpu as pltpu

N_DEV = 32


def _hd_allreduce(x, collective_id):
    rows, cols = x.shape
    rpc = rows // N_DEV
    hns = [16 >> j for j in range(5)]
    offs = [sum(hns[:j]) for j in range(5)]

    def body(x_ref, o_ref, rsrecv, rs_send, rs_recv, ag_send, ag_recv):
        me = lax.axis_index("i")
        o_ref[...] = x_ref[...]

        c0 = [0]
        bs = []
        for j in range(5):
            b = (me >> (4 - j)) & 1
            bs.append(b)
            c0.append(c0[j] + b * hns[j])

        for j in range(5):
            partner = me ^ (16 >> j)
            send_base = (c0[j] + (1 - bs[j]) * hns[j]) * rpc
            size = hns[j] * rpc
            rdma = pltpu.make_async_remote_copy(
                src_ref=o_ref.at[pl.ds(send_base, size), :],
                dst_ref=rsrecv.at[pl.ds(offs[j] * rpc, size), :],
                send_sem=rs_send.at[j],
                recv_sem=rs_recv.at[j],
                device_id=(partner,),
                device_id_type=pl.DeviceIdType.MESH,
            )
            rdma.start()
            rdma.wait()
            keep = pl.ds(c0[j + 1] * rpc, size)
            o_ref[keep, :] = o_ref[keep, :] + rsrecv[pl.ds(offs[j] * rpc, size), :]

        for j in range(4, -1, -1):
            partner = me ^ (16 >> j)
            size = hns[j] * rpc
            seg = pl.ds(c0[j + 1] * rpc, size)
            rdma = pltpu.make_async_remote_copy(
                src_ref=o_ref.at[seg, :],
                dst_ref=o_ref.at[seg, :],
                send_sem=ag_send.at[j],
                recv_sem=ag_recv.at[j],
                device_id=(partner,),
                device_id_type=pl.DeviceIdType.MESH,
            )
            rdma.start()
            rdma.wait()

    return pl.pallas_call(
        body,
        out_shape=jax.ShapeDtypeStruct((rows, cols), jnp.float32),
        in_specs=[pl.BlockSpec(memory_space=pltpu.VMEM)],
        out_specs=pl.BlockSpec(memory_space=pltpu.VMEM),
        scratch_shapes=[
            pltpu.VMEM((31 * rpc, cols), jnp.float32),
            pltpu.SemaphoreType.DMA((5,)),
            pltpu.SemaphoreType.DMA((5,)),
            pltpu.SemaphoreType.DMA((5,)),
            pltpu.SemaphoreType.DMA((5,)),
        ],
        compiler_params=pltpu.CompilerParams(has_side_effects=True),
    )(x)


def _ring_allreduce(x, collective_id):
    rows, cols = x.shape
    chunk = rows // N_DEV

    def body(x_ref, o_ref, rs_buf, ag_buf, rs_send, rs_recv, ag_send, ag_recv):
        me = lax.axis_index("i")
        right = lax.rem(me + 1, N_DEV)

        o_ref[...] = x_ref[...]

        for h in range(N_DEV - 1):
            c_send = lax.rem(me - h + 2 * N_DEV, N_DEV)
            rdma = pltpu.make_async_remote_copy(
                src_ref=o_ref.at[pl.ds(c_send * chunk, chunk), :],
                dst_ref=rs_buf.at[h],
                send_sem=rs_send.at[h],
                recv_sem=rs_recv.at[h],
                device_id=(right,),
                device_id_type=pl.DeviceIdType.MESH,
            )
            rdma.start()
            rdma.wait()
            c_recv = lax.rem(me - h - 1 + 2 * N_DEV, N_DEV)
            sl = pl.ds(c_recv * chunk, chunk)
            o_ref[sl, :] = o_ref[sl, :] + rs_buf[h]

        for h in range(N_DEV - 1):
            c_send = lax.rem(me + 1 - h + 2 * N_DEV, N_DEV)
            rdma = pltpu.make_async_remote_copy(
                src_ref=o_ref.at[pl.ds(c_send * chunk, chunk), :],
                dst_ref=ag_buf.at[h],
                send_sem=ag_send.at[h],
                recv_sem=ag_recv.at[h],
                device_id=(right,),
                device_id_type=pl.DeviceIdType.MESH,
            )
            rdma.start()
            rdma.wait()
            c_recv = lax.rem(me - h + 2 * N_DEV, N_DEV)
            o_ref[pl.ds(c_recv * chunk, chunk), :] = ag_buf[h]

    return pl.pallas_call(
        body,
        out_shape=jax.ShapeDtypeStruct((rows, cols), jnp.float32),
        in_specs=[pl.BlockSpec(memory_space=pltpu.VMEM)],
        out_specs=pl.BlockSpec(memory_space=pltpu.VMEM),
        scratch_shapes=[
            pltpu.VMEM((N_DEV - 1, chunk, cols), jnp.float32),
            pltpu.VMEM((N_DEV - 1, chunk, cols), jnp.float32),
            pltpu.SemaphoreType.DMA((N_DEV - 1,)),
            pltpu.SemaphoreType.DMA((N_DEV - 1,)),
            pltpu.SemaphoreType.DMA((N_DEV - 1,)),
            pltpu.SemaphoreType.DMA((N_DEV - 1,)),
        ],
        compiler_params=pltpu.CompilerParams(has_side_effects=True),
    )(x)


def _ln(h):
    m = h.mean(axis=-1, keepdims=True)
    v = h.var(axis=-1, keepdims=True)
    return (h - m) / jnp.sqrt(v + 1e-5)


def kernel(x, Wq, Wk, Wv, Wo, t_emb, W_mod, W_ff1, W_ff2):
    B, S, D = x.shape
    Dh = 96
    Hloc = Wq.shape[1] // Dh

    x0 = x.astype(jnp.float32)
    mod = t_emb @ W_mod
    sa, sha, ga, sm, shm, gm = jnp.split(mod, 6, axis=-1)

    xm = _ln(x0) * (1.0 + sa[:, None, :]) + sha[:, None, :]

    Q = (xm @ Wq).reshape(B, S, Hloc, Dh)
    K = (xm @ Wk).reshape(B, S, Hloc, Dh)
    V = (xm @ Wv).reshape(B, S, Hloc, Dh)
    s = jnp.einsum("bihd,bjhd->bhij", Q, K) * 0.10206207261596577
    p = jax.nn.softmax(s.astype(jnp.float32), axis=-1)
    o = jnp.einsum("bhij,bjhd->bihd", p, V).reshape(B, S, Hloc * Dh)
    attn_partial = o @ Wo

    attn = _hd_allreduce(attn_partial.reshape(B * S, D), 0).reshape(B, S, D)
    x1 = x0 + ga[:, None, :] * attn

    xm2 = _ln(x1) * (1.0 + sm[:, None, :]) + shm[:, None, :]
    h = xm2 @ W_ff1
    h = h * jax.nn.sigmoid(h)
    ff_partial = h @ W_ff2

    ff = _hd_allreduce(ff_partial.reshape(B * S, D), 1).reshape(B, S, D)
    return x1 + gm[:, None, :] * ff
